# Initial kernel scaffold; baseline (speedup 1.0000x reference)
import jax
import jax.numpy as jnp
from jax.experimental import pallas as pl


def kernel(x, edge_index, W1, b1, W2, b2):
    raise NotImplementedError("write your pallas kernel here")


# speedup vs baseline: 11.3144x; 11.3144x over previous
import functools

import jax
import jax.numpy as jnp
from jax import lax
from jax.experimental import pallas as pl
from jax.experimental.pallas import tpu as pltpu
from jax.experimental.pallas import tpu_sc as plsc

NC = 2
NS = 16
L = 16
D = 128
CHUNK = 128

f32 = jnp.float32
i32 = jnp.int32


def _deg_body(npad, echunks, dst_hbm, out_hbm, hist, ibuf, sem):
    del sem
    cid = lax.axis_index("c")
    sid = lax.axis_index("s")
    wid = cid * NS + sid
    zeros = jnp.zeros((L,), f32)
    ones = jnp.ones((L,), f32)

    def zero_body(i, _):
        hist[pl.ds(pl.multiple_of(i * L, L), L)] = zeros
        return 0
    lax.fori_loop(0, npad // L, zero_body, 0)

    base = wid * (echunks * CHUNK)

    def chunk_body(c, _):
        e0 = pl.multiple_of(base + c * CHUNK, CHUNK)
        pltpu.sync_copy(dst_hbm.at[pl.ds(e0, CHUNK)], ibuf)

        def inner(j, _):
            idx = ibuf[pl.ds(pl.multiple_of(j * L, L), L)]
            plsc.addupdate_scatter(hist, [idx], ones)
            return 0
        lax.fori_loop(0, CHUNK // L, inner, 0)
        return 0
    lax.fori_loop(0, echunks, chunk_body, 0)

    pltpu.sync_copy(hist, out_hbm.at[wid])


def _make_deg(npad, epad):
    echunks = epad // (NC * NS * CHUNK)
    mesh = plsc.VectorSubcoreMesh(core_axis_name="c", subcore_axis_name="s")
    return pl.kernel(
        functools.partial(_deg_body, npad, echunks),
        out_type=jax.ShapeDtypeStruct((NC * NS, npad), f32),
        mesh=mesh,
        scratch_types=[
            pltpu.VMEM((npad,), f32),
            pltpu.VMEM((CHUNK,), i32),
            pltpu.SemaphoreType.DMA,
        ],
        compiler_params=pltpu.CompilerParams(needs_layout_passes=False),
    )


def _msg_body(npad, echunks, tab_hbm, src_hbm, dst_hbm, out_hbm,
              acc_sh, rows, sidx, didx, sem):
    cid = lax.axis_index("c")
    sid = lax.axis_index("s")
    wid = cid * NS + sid
    rows_pt = npad // NS
    zeros = jnp.zeros((L,), f32)

    def zrow(r, _):
        for cc in range(D // L):
            rows[r, pl.ds(cc * L, L)] = zeros
        return 0
    lax.fori_loop(0, CHUNK, zrow, 0)

    r0 = sid * rows_pt
    for k in range(rows_pt // CHUNK):
        pltpu.sync_copy(rows, acc_sh.at[pl.ds(r0 + k * CHUNK, CHUNK)])
    plsc.subcore_barrier()

    base = wid * (echunks * CHUNK)

    def chunk_body(c, _):
        e0 = pl.multiple_of(base + c * CHUNK, CHUNK)
        pltpu.sync_copy(src_hbm.at[pl.ds(e0, CHUNK)], sidx)
        pltpu.sync_copy(dst_hbm.at[pl.ds(e0, CHUNK)], didx)
        pltpu.async_copy(tab_hbm.at[sidx], rows, sem).wait()
        pltpu.sync_copy(rows, acc_sh.at[didx], add=True)
        return 0
    lax.fori_loop(0, echunks, chunk_body, 0)

    plsc.subcore_barrier()
    for k in range(rows_pt // CHUNK):
        pltpu.sync_copy(acc_sh.at[pl.ds(r0 + k * CHUNK, CHUNK)], rows)
        pltpu.sync_copy(rows, out_hbm.at[cid, pl.ds(r0 + k * CHUNK, CHUNK)])


def _make_msg(npad, epad):
    echunks = epad // (NC * NS * CHUNK)
    mesh = plsc.VectorSubcoreMesh(core_axis_name="c", subcore_axis_name="s")
    return pl.kernel(
        functools.partial(_msg_body, npad, echunks),
        out_type=jax.ShapeDtypeStruct((NC, npad, D), f32),
        mesh=mesh,
        scratch_types=[
            pltpu.VMEM_SHARED((npad, D), f32),
            pltpu.VMEM((CHUNK, D), f32),
            pltpu.VMEM((CHUNK,), i32),
            pltpu.VMEM((CHUNK,), i32),
            pltpu.SemaphoreType.DMA,
        ],
        compiler_params=pltpu.CompilerParams(needs_layout_passes=False),
    )


R = 256


def _dis_from_hist(hb):
    deg = jnp.sum(hb[...], axis=0) + 1.0
    return lax.rsqrt(deg)[:, None]


def _tc1_body(xb, wb, hb, hob):
    h = jnp.dot(xb[...], wb[...], preferred_element_type=f32)
    hob[...] = h * _dis_from_hist(hb)


def _tc2_body(mb, hb, histb, wb, bb, ob):
    dis = _dis_from_hist(histb)
    m = mb[...]
    z = (m[0] + m[1] + hb[...]) * dis + bb[...]
    z = jnp.maximum(z, 0.0)
    ob[...] = jnp.dot(z, wb[...], preferred_element_type=f32) * dis


def _tc3_body(mb, hb, histb, bb, ob):
    dis = _dis_from_hist(histb)
    m = mb[...]
    ob[...] = (m[0] + m[1] + hb[...]) * dis + bb[...]


def _make_tc(npad):
    nb = npad // R
    row = pl.BlockSpec((R, D), lambda i: (i, 0))
    full_w = pl.BlockSpec((D, D), lambda i: (0, 0))
    bias = pl.BlockSpec((1, D), lambda i: (0, 0))
    msg = pl.BlockSpec((NC, R, D), lambda i: (0, i, 0))
    hist = pl.BlockSpec((NC * NS, R), lambda i: (0, i))

    tc1 = pl.pallas_call(
        _tc1_body,
        grid=(nb,),
        in_specs=[row, full_w, hist],
        out_specs=row,
        out_shape=jax.ShapeDtypeStruct((npad, D), f32),
    )
    tc2 = pl.pallas_call(
        _tc2_body,
        grid=(nb,),
        in_specs=[msg, row, hist, full_w, bias],
        out_specs=row,
        out_shape=jax.ShapeDtypeStruct((npad, D), f32),
    )
    tc3 = pl.pallas_call(
        _tc3_body,
        grid=(nb,),
        in_specs=[msg, row, hist, bias],
        out_specs=row,
        out_shape=jax.ShapeDtypeStruct((npad, D), f32),
    )
    return tc1, tc2, tc3


def kernel(x, edge_index, W1, b1, W2, b2):
    n, d = x.shape
    e = edge_index.shape[1]
    assert d == D
    nstep = NS * CHUNK
    npad = pl.cdiv(n + 1, nstep) * nstep
    estep = NC * NS * CHUNK
    epad = pl.cdiv(e, estep) * estep

    ei = edge_index.astype(i32)
    pad = jnp.full((epad - e,), n, i32)
    src = jnp.concatenate([ei[0], pad])
    dst = jnp.concatenate([ei[1], pad])
    xp = jnp.pad(x, ((0, npad - n), (0, 0)))

    deg_call = _make_deg(npad, epad)
    msg_call = _make_msg(npad, epad)
    tc1, tc2, tc3 = _make_tc(npad)

    hists = deg_call(dst)
    h1p = tc1(xp, W1, hists)
    m1 = msg_call(h1p, src, dst)
    h2p = tc2(m1, h1p, hists, W2, b1.reshape(1, D))
    m2 = msg_call(h2p, src, dst)
    outp = tc3(m2, h2p, hists, b2.reshape(1, D))
    return outp[:n]
